# baseline (speedup 1.0000x reference)
import functools

import jax
import jax.numpy as jnp
from jax import lax
from jax.experimental import pallas as pl
from jax.experimental.pallas import tpu as pltpu
from jax.experimental.pallas import tpu_sc as plsc

N = 10000
E = 320000
D = 128
RD = 64
FD = D + RD
OD = 128

NC = 2
NS = 16
NW = NC * NS
KC = 128
NCH = E // KC
NCMAX = NCH // NW + 1

TB = (0, 5, 25, 45, 65, 75)
CS = (160, 640, 640, 640, 320, 100)
NSLICE = len(CS)


@functools.lru_cache(maxsize=None)
def _mesh():
  return plsc.VectorSubcoreMesh(core_axis_name="c", subcore_axis_name="s",
                                num_cores=NC, num_subcores=NS)


def _make_gather_body(s):
  t0 = TB[s]
  t1s = TB[s + 1] if s + 1 < NSLICE else None
  base = t0 * NW * KC

  def gather_body(x_hbm, pos4_hbm, srcw_hbm, dstw_hbm,
                  xj_hbm, pf_hbm,
                  sidx, didx, posv, xbuf, pbuf, sem):
    cid = lax.axis_index("c")
    sid = lax.axis_index("s")
    wid = sid * NC + cid
    t1 = (78 + jnp.where(wid < NCH - 78 * NW, 1, 0)) if t1s is None else t1s
    pltpu.sync_copy(srcw_hbm.at[wid], sidx)
    pltpu.sync_copy(dstw_hbm.at[wid], didx)
    pltpu.sync_copy(pos4_hbm, posv)

    def body(t, carry):
      off = (t * NW + wid) * KC - base
      cp = pltpu.async_copy(x_hbm.at[sidx.at[t]], xbuf, sem)
      for g in range(KC // 16):
        svec4 = sidx[t, pl.ds(g * 16, 16)] * 4
        dvec4 = didx[t, pl.ds(g * 16, 16)] * 4
        d2 = None
        for k in range(3):
          pjc = plsc.load_gather(posv, [svec4 + k])
          pic = plsc.load_gather(posv, [dvec4 + k])
          vc = pic - pjc
          d2 = vc * vc if d2 is None else d2 + vc * vc
          pbuf[k, pl.ds(g * 16, 16)] = pic
          pbuf[k + 3, pl.ds(g * 16, 16)] = pjc
        pbuf[6, pl.ds(g * 16, 16)] = d2
      cp.wait()
      pltpu.sync_copy(xbuf, xj_hbm.at[pl.ds(off, KC)])
      pltpu.sync_copy(pbuf, pf_hbm.at[:, pl.ds(off, KC)])
      return carry

    lax.fori_loop(t0, t1, body, 0)

  return gather_body


@functools.lru_cache(maxsize=None)
def _sc_gather_kernel(s):
  es = CS[s] * KC
  return pl.kernel(
      _make_gather_body(s),
      out_type=(
          jax.ShapeDtypeStruct((es, D), jnp.float32),
          jax.ShapeDtypeStruct((8, es), jnp.float32),
      ),
      mesh=_mesh(),
      scratch_types=[
          pltpu.VMEM((NCMAX, KC), jnp.int32),
          pltpu.VMEM((NCMAX, KC), jnp.int32),
          pltpu.VMEM((N * 4,), jnp.float32),
          pltpu.VMEM((KC, D), jnp.float32),
          pltpu.VMEM((8, KC), jnp.float32),
          pltpu.SemaphoreType.DMA,
      ],
      compiler_params=pltpu.CompilerParams(needs_layout_passes=False),
  )


def _sc_gather(x, pos4, srcw, dstw, s):
  return _sc_gather_kernel(s)(x, pos4, srcw, dstw)


def _make_scatter_body(s):
  t0 = TB[s]
  t1s = TB[s + 1] if s + 1 < NSLICE else None
  base = t0 * NW * KC

  def scatter_body(msg_hbm, dstw_hbm, z_hbm, p0_hbm, p1_hbm,
                   didx, buf, shared, sem):
    cid = lax.axis_index("c")
    sid = lax.axis_index("s")
    wid = sid * NC + cid
    t1 = (78 + jnp.where(wid < NCH - 78 * NW, 1, 0)) if t1s is None else t1s
    @pl.when(sid < NS - 1)
    def _():
      pltpu.sync_copy(z_hbm, shared.at[pl.ds(sid * 640, 640)])

    @pl.when(sid == NS - 1)
    def _():
      pltpu.sync_copy(z_hbm.at[pl.ds(0, 400)], shared.at[pl.ds(9600, 400)])

    pltpu.sync_copy(dstw_hbm.at[wid], didx)
    plsc.subcore_barrier()

    pltpu.async_copy(msg_hbm.at[pl.ds((t0 * NW + wid) * KC - base, KC)],
                     buf.at[t0 % 2], sem)

    def body(t, carry):
      @pl.when(t + 1 < t1)
      def _():
        off1 = ((t + 1) * NW + wid) * KC - base
        pltpu.async_copy(msg_hbm.at[pl.ds(off1, KC)], buf.at[(t + 1) % 2],
                         sem)

      pltpu.make_async_copy(msg_hbm.at[pl.ds(0, KC)], buf.at[t % 2],
                            sem).wait()
      pltpu.sync_copy(buf.at[t % 2], shared.at[didx.at[t]], add=True)
      return carry

    lax.fori_loop(t0, t1, body, 0)
    plsc.subcore_barrier()

    @pl.when(cid == 0)
    def _():
      @pl.when(sid < NS - 1)
      def _():
        pltpu.sync_copy(shared.at[pl.ds(sid * 640, 640)],
                        p0_hbm.at[pl.ds(sid * 640, 640)])

      @pl.when(sid == NS - 1)
      def _():
        pltpu.sync_copy(shared.at[pl.ds(9600, 400)],
                        p0_hbm.at[pl.ds(9600, 400)])

    @pl.when(cid == 1)
    def _():
      @pl.when(sid < NS - 1)
      def _():
        pltpu.sync_copy(shared.at[pl.ds(sid * 640, 640)],
                        p1_hbm.at[pl.ds(sid * 640, 640)])

      @pl.when(sid == NS - 1)
      def _():
        pltpu.sync_copy(shared.at[pl.ds(9600, 400)],
                        p1_hbm.at[pl.ds(9600, 400)])

  return scatter_body


@functools.lru_cache(maxsize=None)
def _sc_scatter_kernel(s):
  return pl.kernel(
      _make_scatter_body(s),
      out_type=(
          jax.ShapeDtypeStruct((N, OD), jnp.float32),
          jax.ShapeDtypeStruct((N, OD), jnp.float32),
      ),
      mesh=_mesh(),
      scratch_types=[
          pltpu.VMEM((NCMAX, KC), jnp.int32),
          pltpu.VMEM((2, KC, OD), jnp.float32),
          pltpu.VMEM_SHARED((N, OD), jnp.float32),
          pltpu.SemaphoreType.DMA,
      ],
      compiler_params=pltpu.CompilerParams(needs_layout_passes=False),
  )


def _sc_scatter(msg, dstw, z, s):
  return _sc_scatter_kernel(s)(msg, dstw, z)


B_EDGE = 2560


def _edge_body(xj_ref, pf_ref, w65_ref, wpd_ref, bp_ref,
               wax_ref, war_ref, ba_ref, wgx_ref, wgr_ref, ones_ref,
               msg_ref):
  xj = xj_ref[...]
  pf7 = pf_ref[...][:7, :]
  lin = lax.dot_general(pf7, w65_ref[...], (((0,), (0,)), ((), ())),
                        preferred_element_type=jnp.float32)
  dij = jnp.sqrt(lin[:, RD:RD + 1] + 1e-12)
  rij = jnp.maximum(lin[:, :RD] + dij * wpd_ref[...] + bp_ref[...], 0.0)
  xj16 = xj.astype(jnp.bfloat16)
  rij16 = rij.astype(jnp.bfloat16)
  g = jnp.dot(xj16, wax_ref[...], preferred_element_type=jnp.float32)
  g += jnp.dot(rij16, war_ref[...], preferred_element_type=jnp.float32)
  g = jnp.maximum(g + ba_ref[...], 0.0)
  eg16 = jnp.exp(g).astype(jnp.bfloat16)
  denom = jnp.dot(eg16, ones_ref[...], preferred_element_type=jnp.float32)
  o = jnp.dot(eg16[:, :D] * xj16, wgx_ref[...],
              preferred_element_type=jnp.float32)
  o += jnp.dot(eg16[:, D:] * rij16, wgr_ref[...],
               preferred_element_type=jnp.float32)
  msg_ref[...] = o * (1.0 / denom)


def _tc_edge(xj, pf, w65, wpd, bp2, wax, war, ba2, wgx, wgr, ones):
  es = xj.shape[0]
  grid = (es // B_EDGE,)
  full = lambda shape: pl.BlockSpec(shape, lambda i: (0, 0))
  return pl.pallas_call(
      _edge_body,
      grid=grid,
      in_specs=[
          pl.BlockSpec((B_EDGE, D), lambda i: (i, 0)),
          pl.BlockSpec((8, B_EDGE), lambda i: (0, i)),
          full((7, RD + 1)),
          full((1, RD)),
          full((1, RD)),
          full((D, FD)),
          full((RD, FD)),
          full((1, FD)),
          full((D, OD)),
          full((RD, OD)),
          full((FD, 1)),
      ],
      out_specs=pl.BlockSpec((B_EDGE, OD), lambda i: (i, 0)),
      out_shape=jax.ShapeDtypeStruct((es, OD), jnp.float32),
  )(xj, pf, w65, wpd, bp2, wax, war, ba2, wgx, wgr, ones)


B_OUT = 2000


def _out_body(*refs):
  ps = refs[:-2]
  bg_ref = refs[-2]
  out_ref = refs[-1]
  acc = ps[0][...]
  for r in ps[1:]:
    acc += r[...]
  out_ref[...] = jnp.maximum(acc + bg_ref[...], 0.0)


def _tc_out(partials, bg2):
  grid = (N // B_OUT,)
  return pl.pallas_call(
      _out_body,
      grid=grid,
      in_specs=[pl.BlockSpec((B_OUT, OD), lambda i: (i, 0))
                for _ in partials] + [pl.BlockSpec((1, OD), lambda i: (0, 0))],
      out_specs=pl.BlockSpec((B_OUT, OD), lambda i: (i, 0)),
      out_shape=jax.ShapeDtypeStruct((N, OD), jnp.float32),
  )(*partials, bg2)


def kernel(x, pos, edge_index, Wp, bp, Wa, ba, Wg, bg):
  src = edge_index[0]
  dst = edge_index[1]
  pos4 = jnp.pad(pos, ((0, 0), (0, 1)))
  ei_pad = jnp.pad(edge_index.reshape(2, NCH, KC),
                   ((0, 0), (0, NCMAX * NW - NCH), (0, 0)))
  ei_w = ei_pad.reshape(2, NCMAX, NW, KC).transpose(0, 2, 1, 3)
  srcw = ei_w[0]
  dstw = ei_w[1]

  w6 = jnp.concatenate([Wp[0:3] + Wp[6:9], Wp[3:6] - Wp[6:9]], axis=0)
  d2col = jnp.concatenate([jnp.zeros((6, 1), jnp.float32),
                           jnp.ones((1, 1), jnp.float32)], axis=0)
  w65 = jnp.concatenate([jnp.pad(w6, ((0, 1), (0, 0))), d2col],
                        axis=1)
  wpd = Wp[9:10]
  bf = jnp.bfloat16
  z = jnp.zeros((640, OD), jnp.float32)
  pos4f = pos4.reshape(-1)

  partials = []
  for s in range(NSLICE):
    xj, pf = _sc_gather(x, pos4f, srcw, dstw, s)
    msg = _tc_edge(xj, pf, w65, wpd, bp.reshape(1, RD),
                   Wa[:D].astype(bf), Wa[D:].astype(bf), ba.reshape(1, FD),
                   Wg[:D].astype(bf), Wg[D:].astype(bf),
                   jnp.ones((FD, 1), bf))
    p0, p1 = _sc_scatter(msg, dstw, z, s)
    partials += [p0, p1]

  return _tc_out(partials, bg.reshape(1, OD))


# speedup vs baseline: 1.0740x; 1.0740x over previous
import functools

import jax
import jax.numpy as jnp
from jax import lax
from jax.experimental import pallas as pl
from jax.experimental.pallas import tpu as pltpu
from jax.experimental.pallas import tpu_sc as plsc

N = 10000
E = 320000
D = 128
RD = 64
FD = D + RD
OD = 128

NC = 2
NS = 16
NW = NC * NS
KC = 128
NCH = E // KC
NCMAX = NCH // NW + 1

TB = (0, 5, 25, 45, 65)
CS = (160, 640, 640, 640, 420)
NSLICE = len(CS)


@functools.lru_cache(maxsize=None)
def _mesh():
  return plsc.VectorSubcoreMesh(core_axis_name="c", subcore_axis_name="s",
                                num_cores=NC, num_subcores=NS)


def _make_gather_body(s):
  t0 = TB[s]
  t1s = TB[s + 1] if s + 1 < NSLICE else None
  base = t0 * NW * KC

  def gather_body(x_hbm, pos4_hbm, srcw_hbm, dstw_hbm,
                  xj_hbm, pf_hbm,
                  sidx, didx, posv, xbuf, pbuf, sem):
    cid = lax.axis_index("c")
    sid = lax.axis_index("s")
    wid = sid * NC + cid
    t1 = (78 + jnp.where(wid < NCH - 78 * NW, 1, 0)) if t1s is None else t1s
    pltpu.sync_copy(srcw_hbm.at[wid], sidx)
    pltpu.sync_copy(dstw_hbm.at[wid], didx)
    pltpu.sync_copy(pos4_hbm, posv)

    def body(t, carry):
      off = (t * NW + wid) * KC - base
      cp = pltpu.async_copy(x_hbm.at[sidx.at[t]], xbuf, sem)
      for g in range(KC // 16):
        svec4 = sidx[t, pl.ds(g * 16, 16)] * 4
        dvec4 = didx[t, pl.ds(g * 16, 16)] * 4
        d2 = None
        for k in range(3):
          pjc = plsc.load_gather(posv, [svec4 + k])
          pic = plsc.load_gather(posv, [dvec4 + k])
          vc = pic - pjc
          d2 = vc * vc if d2 is None else d2 + vc * vc
          pbuf[k, pl.ds(g * 16, 16)] = pic
          pbuf[k + 3, pl.ds(g * 16, 16)] = pjc
        pbuf[6, pl.ds(g * 16, 16)] = d2
      cp.wait()
      pltpu.sync_copy(xbuf, xj_hbm.at[pl.ds(off, KC)])
      pltpu.sync_copy(pbuf, pf_hbm.at[:, pl.ds(off, KC)])
      return carry

    lax.fori_loop(t0, t1, body, 0)

  return gather_body


@functools.lru_cache(maxsize=None)
def _sc_gather_kernel(s):
  es = CS[s] * KC
  return pl.kernel(
      _make_gather_body(s),
      out_type=(
          jax.ShapeDtypeStruct((es, D), jnp.float32),
          jax.ShapeDtypeStruct((8, es), jnp.float32),
      ),
      mesh=_mesh(),
      scratch_types=[
          pltpu.VMEM((NCMAX, KC), jnp.int32),
          pltpu.VMEM((NCMAX, KC), jnp.int32),
          pltpu.VMEM((N * 4,), jnp.float32),
          pltpu.VMEM((KC, D), jnp.float32),
          pltpu.VMEM((8, KC), jnp.float32),
          pltpu.SemaphoreType.DMA,
      ],
      compiler_params=pltpu.CompilerParams(needs_layout_passes=False),
  )


def _sc_gather(x, pos4, srcw, dstw, s):
  return _sc_gather_kernel(s)(x, pos4, srcw, dstw)


def _make_scatter_body(s):
  t0 = TB[s]
  t1s = TB[s + 1] if s + 1 < NSLICE else None
  base = t0 * NW * KC

  def scatter_body(msg_hbm, dstw_hbm, q0_hbm, q1_hbm, p0_hbm, p1_hbm,
                   didx, buf, shared, sem):
    cid = lax.axis_index("c")
    sid = lax.axis_index("s")
    wid = sid * NC + cid
    t1 = (78 + jnp.where(wid < NCH - 78 * NW, 1, 0)) if t1s is None else t1s
    def seed(qq_hbm):
      @pl.when(sid < NS - 1)
      def _():
        pltpu.sync_copy(qq_hbm.at[pl.ds(sid * 640, 640)],
                        shared.at[pl.ds(sid * 640, 640)])

      @pl.when(sid == NS - 1)
      def _():
        pltpu.sync_copy(qq_hbm.at[pl.ds(9600, 400)],
                        shared.at[pl.ds(9600, 400)])

    @pl.when(cid == 0)
    def _():
      seed(q0_hbm)

    @pl.when(cid == 1)
    def _():
      seed(q1_hbm)

    pltpu.sync_copy(dstw_hbm.at[wid], didx)
    plsc.subcore_barrier()

    pltpu.async_copy(msg_hbm.at[pl.ds((t0 * NW + wid) * KC - base, KC)],
                     buf.at[t0 % 2], sem)

    def body(t, carry):
      @pl.when(t + 1 < t1)
      def _():
        off1 = ((t + 1) * NW + wid) * KC - base
        pltpu.async_copy(msg_hbm.at[pl.ds(off1, KC)], buf.at[(t + 1) % 2],
                         sem)

      pltpu.make_async_copy(msg_hbm.at[pl.ds(0, KC)], buf.at[t % 2],
                            sem).wait()
      pltpu.sync_copy(buf.at[t % 2], shared.at[didx.at[t]], add=True)
      return carry

    lax.fori_loop(t0, t1, body, 0)
    plsc.subcore_barrier()

    @pl.when(cid == 0)
    def _():
      @pl.when(sid < NS - 1)
      def _():
        pltpu.sync_copy(shared.at[pl.ds(sid * 640, 640)],
                        p0_hbm.at[pl.ds(sid * 640, 640)])

      @pl.when(sid == NS - 1)
      def _():
        pltpu.sync_copy(shared.at[pl.ds(9600, 400)],
                        p0_hbm.at[pl.ds(9600, 400)])

    @pl.when(cid == 1)
    def _():
      @pl.when(sid < NS - 1)
      def _():
        pltpu.sync_copy(shared.at[pl.ds(sid * 640, 640)],
                        p1_hbm.at[pl.ds(sid * 640, 640)])

      @pl.when(sid == NS - 1)
      def _():
        pltpu.sync_copy(shared.at[pl.ds(9600, 400)],
                        p1_hbm.at[pl.ds(9600, 400)])

  return scatter_body


@functools.lru_cache(maxsize=None)
def _sc_scatter_kernel(s):
  return pl.kernel(
      _make_scatter_body(s),
      out_type=(
          jax.ShapeDtypeStruct((N, OD), jnp.float32),
          jax.ShapeDtypeStruct((N, OD), jnp.float32),
      ),
      mesh=_mesh(),
      scratch_types=[
          pltpu.VMEM((NCMAX, KC), jnp.int32),
          pltpu.VMEM((2, KC, OD), jnp.float32),
          pltpu.VMEM_SHARED((N, OD), jnp.float32),
          pltpu.SemaphoreType.DMA,
      ],
      compiler_params=pltpu.CompilerParams(needs_layout_passes=False),
  )


def _sc_scatter(msg, dstw, q0, q1, s):
  return _sc_scatter_kernel(s)(msg, dstw, q0, q1)


B_EDGE = 2560


def _edge_body(xj_ref, pf_ref, w65_ref, wpd_ref, bp_ref,
               wax_ref, war_ref, ba_ref, wgx_ref, wgr_ref, ones_ref,
               msg_ref):
  xj = xj_ref[...]
  pf7 = pf_ref[...][:7, :]
  lin = lax.dot_general(pf7, w65_ref[...], (((0,), (0,)), ((), ())),
                        preferred_element_type=jnp.float32)
  dij = jnp.sqrt(lin[:, RD:RD + 1] + 1e-12)
  rij = jnp.maximum(lin[:, :RD] + dij * wpd_ref[...] + bp_ref[...], 0.0)
  xj16 = xj.astype(jnp.bfloat16)
  rij16 = rij.astype(jnp.bfloat16)
  g = jnp.dot(xj16, wax_ref[...], preferred_element_type=jnp.float32)
  g += jnp.dot(rij16, war_ref[...], preferred_element_type=jnp.float32)
  g = jnp.maximum(g + ba_ref[...], 0.0)
  eg16 = jnp.exp(g).astype(jnp.bfloat16)
  denom = jnp.dot(eg16, ones_ref[...], preferred_element_type=jnp.float32)
  o = jnp.dot(eg16[:, :D] * xj16, wgx_ref[...],
              preferred_element_type=jnp.float32)
  o += jnp.dot(eg16[:, D:] * rij16, wgr_ref[...],
               preferred_element_type=jnp.float32)
  msg_ref[...] = o * (1.0 / denom)


def _tc_edge(xj, pf, w65, wpd, bp2, wax, war, ba2, wgx, wgr, ones):
  es = xj.shape[0]
  grid = (es // B_EDGE,)
  full = lambda shape: pl.BlockSpec(shape, lambda i: (0, 0))
  return pl.pallas_call(
      _edge_body,
      grid=grid,
      in_specs=[
          pl.BlockSpec((B_EDGE, D), lambda i: (i, 0)),
          pl.BlockSpec((8, B_EDGE), lambda i: (0, i)),
          full((7, RD + 1)),
          full((1, RD)),
          full((1, RD)),
          full((D, FD)),
          full((RD, FD)),
          full((1, FD)),
          full((D, OD)),
          full((RD, OD)),
          full((FD, 1)),
      ],
      out_specs=pl.BlockSpec((B_EDGE, OD), lambda i: (i, 0)),
      out_shape=jax.ShapeDtypeStruct((es, OD), jnp.float32),
  )(xj, pf, w65, wpd, bp2, wax, war, ba2, wgx, wgr, ones)


B_OUT = 2000


def _out_body(*refs):
  ps = refs[:-2]
  bg_ref = refs[-2]
  out_ref = refs[-1]
  acc = ps[0][...]
  for r in ps[1:]:
    acc += r[...]
  out_ref[...] = jnp.maximum(acc + bg_ref[...], 0.0)


def _tc_out(partials, bg2):
  grid = (N // B_OUT,)
  return pl.pallas_call(
      _out_body,
      grid=grid,
      in_specs=[pl.BlockSpec((B_OUT, OD), lambda i: (i, 0))
                for _ in partials] + [pl.BlockSpec((1, OD), lambda i: (0, 0))],
      out_specs=pl.BlockSpec((B_OUT, OD), lambda i: (i, 0)),
      out_shape=jax.ShapeDtypeStruct((N, OD), jnp.float32),
  )(*partials, bg2)


def kernel(x, pos, edge_index, Wp, bp, Wa, ba, Wg, bg):
  src = edge_index[0]
  dst = edge_index[1]
  pos4 = jnp.pad(pos, ((0, 0), (0, 1)))
  ei_pad = jnp.pad(edge_index.reshape(2, NCH, KC),
                   ((0, 0), (0, NCMAX * NW - NCH), (0, 0)))
  ei_w = ei_pad.reshape(2, NCMAX, NW, KC).transpose(0, 2, 1, 3)
  srcw = ei_w[0]
  dstw = ei_w[1]

  w6 = jnp.concatenate([Wp[0:3] + Wp[6:9], Wp[3:6] - Wp[6:9]], axis=0)
  d2col = jnp.concatenate([jnp.zeros((6, 1), jnp.float32),
                           jnp.ones((1, 1), jnp.float32)], axis=0)
  w65 = jnp.concatenate([jnp.pad(w6, ((0, 1), (0, 0))), d2col],
                        axis=1)
  wpd = Wp[9:10]
  bf = jnp.bfloat16
  pos4f = pos4.reshape(-1)

  q0 = jnp.zeros((N, OD), jnp.float32)
  q1 = q0
  for s in range(NSLICE):
    xj, pf = _sc_gather(x, pos4f, srcw, dstw, s)
    msg = _tc_edge(xj, pf, w65, wpd, bp.reshape(1, RD),
                   Wa[:D].astype(bf), Wa[D:].astype(bf), ba.reshape(1, FD),
                   Wg[:D].astype(bf), Wg[D:].astype(bf),
                   jnp.ones((FD, 1), bf))
    q0, q1 = _sc_scatter(msg, dstw, q0, q1, s)

  return _tc_out([q0, q1], bg.reshape(1, OD))


# speedup vs baseline: 1.1075x; 1.0312x over previous
import functools

import jax
import jax.numpy as jnp
from jax import lax
from jax.experimental import pallas as pl
from jax.experimental.pallas import tpu as pltpu
from jax.experimental.pallas import tpu_sc as plsc

N = 10000
E = 320000
D = 128
RD = 64
FD = D + RD
OD = 128

NC = 2
NS = 16
NW = NC * NS
KC = 128
NCH = E // KC
NCMAX = NCH // NW + 1

TB = (0, 5, 25, 45, 65)
CS = (160, 640, 640, 640, 420)
NSLICE = len(CS)


@functools.lru_cache(maxsize=None)
def _mesh():
  return plsc.VectorSubcoreMesh(core_axis_name="c", subcore_axis_name="s",
                                num_cores=NC, num_subcores=NS)


def _make_gather_body(s):
  t0 = TB[s]
  t1s = TB[s + 1] if s + 1 < NSLICE else None
  base = t0 * NW * KC

  def gather_body(x_hbm, pos4_hbm, srcw_hbm, dstw_hbm,
                  xj_hbm, pf_hbm,
                  sidx, didx, posv, xbuf, pbuf, semg, semw):
    cid = lax.axis_index("c")
    sid = lax.axis_index("s")
    wid = sid * NC + cid
    t1 = (78 + jnp.where(wid < NCH - 78 * NW, 1, 0)) if t1s is None else t1s
    pltpu.sync_copy(srcw_hbm.at[wid], sidx)
    pltpu.sync_copy(dstw_hbm.at[wid], didx)
    pltpu.sync_copy(pos4_hbm, posv)
    pltpu.async_copy(x_hbm.at[sidx.at[t0]], xbuf.at[t0 % 2], semg)

    def body(t, carry):
      p = t % 2
      off = (t * NW + wid) * KC - base
      for g in range(KC // 16):
        svec4 = sidx[t, pl.ds(g * 16, 16)] * 4
        dvec4 = didx[t, pl.ds(g * 16, 16)] * 4
        d2 = None
        for k in range(3):
          pjc = plsc.load_gather(posv, [svec4 + k])
          pic = plsc.load_gather(posv, [dvec4 + k])
          vc = pic - pjc
          d2 = vc * vc if d2 is None else d2 + vc * vc
          pbuf[p, k, pl.ds(g * 16, 16)] = pic
          pbuf[p, k + 3, pl.ds(g * 16, 16)] = pjc
        pbuf[p, 6, pl.ds(g * 16, 16)] = d2
      pltpu.make_async_copy(x_hbm.at[sidx.at[t]], xbuf.at[p], semg).wait()
      pltpu.async_copy(xbuf.at[p], xj_hbm.at[pl.ds(off, KC)], semw)
      pltpu.async_copy(pbuf.at[p], pf_hbm.at[:, pl.ds(off, KC)], semw)

      @pl.when(t > t0)
      def _():
        offp = ((t - 1) * NW + wid) * KC - base
        pltpu.make_async_copy(xbuf.at[1 - p],
                              xj_hbm.at[pl.ds(offp, KC)], semw).wait()
        pltpu.make_async_copy(pbuf.at[1 - p],
                              pf_hbm.at[:, pl.ds(offp, KC)], semw).wait()

      @pl.when(t + 1 < t1)
      def _():
        pltpu.async_copy(x_hbm.at[sidx.at[t + 1]], xbuf.at[1 - p], semg)

      return carry

    lax.fori_loop(t0, t1, body, 0)
    pf = (t1 - 1) % 2
    offl = ((t1 - 1) * NW + wid) * KC - base
    pltpu.make_async_copy(xbuf.at[pf], xj_hbm.at[pl.ds(offl, KC)],
                          semw).wait()
    pltpu.make_async_copy(pbuf.at[pf], pf_hbm.at[:, pl.ds(offl, KC)],
                          semw).wait()

  return gather_body


@functools.lru_cache(maxsize=None)
def _sc_gather_kernel(s):
  es = CS[s] * KC
  return pl.kernel(
      _make_gather_body(s),
      out_type=(
          jax.ShapeDtypeStruct((es, D), jnp.float32),
          jax.ShapeDtypeStruct((8, es), jnp.float32),
      ),
      mesh=_mesh(),
      scratch_types=[
          pltpu.VMEM((NCMAX, KC), jnp.int32),
          pltpu.VMEM((NCMAX, KC), jnp.int32),
          pltpu.VMEM((N * 4,), jnp.float32),
          pltpu.VMEM((2, KC, D), jnp.float32),
          pltpu.VMEM((2, 8, KC), jnp.float32),
          pltpu.SemaphoreType.DMA,
          pltpu.SemaphoreType.DMA,
      ],
      compiler_params=pltpu.CompilerParams(needs_layout_passes=False),
  )


def _sc_gather(x, pos4, srcw, dstw, s):
  return _sc_gather_kernel(s)(x, pos4, srcw, dstw)


def _make_scatter_body(s):
  t0 = TB[s]
  t1s = TB[s + 1] if s + 1 < NSLICE else None
  base = t0 * NW * KC

  def scatter_body(msg_hbm, dstw_hbm, q0_hbm, q1_hbm, p0_hbm, p1_hbm,
                   didx, buf, shared, sem):
    cid = lax.axis_index("c")
    sid = lax.axis_index("s")
    wid = sid * NC + cid
    t1 = (78 + jnp.where(wid < NCH - 78 * NW, 1, 0)) if t1s is None else t1s
    def seed(qq_hbm):
      @pl.when(sid < NS - 1)
      def _():
        pltpu.sync_copy(qq_hbm.at[pl.ds(sid * 640, 640)],
                        shared.at[pl.ds(sid * 640, 640)])

      @pl.when(sid == NS - 1)
      def _():
        pltpu.sync_copy(qq_hbm.at[pl.ds(9600, 400)],
                        shared.at[pl.ds(9600, 400)])

    @pl.when(cid == 0)
    def _():
      seed(q0_hbm)

    @pl.when(cid == 1)
    def _():
      seed(q1_hbm)

    pltpu.sync_copy(dstw_hbm.at[wid], didx)
    plsc.subcore_barrier()

    pltpu.async_copy(msg_hbm.at[pl.ds((t0 * NW + wid) * KC - base, KC)],
                     buf.at[t0 % 2], sem)

    def body(t, carry):
      @pl.when(t + 1 < t1)
      def _():
        off1 = ((t + 1) * NW + wid) * KC - base
        pltpu.async_copy(msg_hbm.at[pl.ds(off1, KC)], buf.at[(t + 1) % 2],
                         sem)

      pltpu.make_async_copy(msg_hbm.at[pl.ds(0, KC)], buf.at[t % 2],
                            sem).wait()
      pltpu.sync_copy(buf.at[t % 2], shared.at[didx.at[t]], add=True)
      return carry

    lax.fori_loop(t0, t1, body, 0)
    plsc.subcore_barrier()

    @pl.when(cid == 0)
    def _():
      @pl.when(sid < NS - 1)
      def _():
        pltpu.sync_copy(shared.at[pl.ds(sid * 640, 640)],
                        p0_hbm.at[pl.ds(sid * 640, 640)])

      @pl.when(sid == NS - 1)
      def _():
        pltpu.sync_copy(shared.at[pl.ds(9600, 400)],
                        p0_hbm.at[pl.ds(9600, 400)])

    @pl.when(cid == 1)
    def _():
      @pl.when(sid < NS - 1)
      def _():
        pltpu.sync_copy(shared.at[pl.ds(sid * 640, 640)],
                        p1_hbm.at[pl.ds(sid * 640, 640)])

      @pl.when(sid == NS - 1)
      def _():
        pltpu.sync_copy(shared.at[pl.ds(9600, 400)],
                        p1_hbm.at[pl.ds(9600, 400)])

  return scatter_body


@functools.lru_cache(maxsize=None)
def _sc_scatter_kernel(s):
  return pl.kernel(
      _make_scatter_body(s),
      out_type=(
          jax.ShapeDtypeStruct((N, OD), jnp.float32),
          jax.ShapeDtypeStruct((N, OD), jnp.float32),
      ),
      mesh=_mesh(),
      scratch_types=[
          pltpu.VMEM((NCMAX, KC), jnp.int32),
          pltpu.VMEM((2, KC, OD), jnp.float32),
          pltpu.VMEM_SHARED((N, OD), jnp.float32),
          pltpu.SemaphoreType.DMA,
      ],
      compiler_params=pltpu.CompilerParams(needs_layout_passes=False),
  )


def _sc_scatter(msg, dstw, q0, q1, s):
  return _sc_scatter_kernel(s)(msg, dstw, q0, q1)


B_EDGE = 2560


def _edge_body(xj_ref, pf_ref, w65_ref, wpd_ref, bp_ref,
               wax_ref, war_ref, ba_ref, wgx_ref, wgr_ref, ones_ref,
               msg_ref):
  xj16 = xj_ref[...].astype(jnp.bfloat16)
  pf7 = pf_ref[...][:7, :]
  lin = lax.dot_general(pf7, w65_ref[...], (((0,), (0,)), ((), ())),
                        preferred_element_type=jnp.float32)
  dij = jnp.sqrt(lin[:, RD:RD + 1] + 1e-12)
  rij = jnp.maximum(lin[:, :RD] + dij * wpd_ref[...] + bp_ref[...], 0.0)
  rij16 = rij.astype(jnp.bfloat16)
  g = jnp.dot(xj16, wax_ref[...], preferred_element_type=jnp.float32)
  g += jnp.dot(rij16, war_ref[...], preferred_element_type=jnp.float32)
  g = jnp.maximum(g + ba_ref[...], 0.0)
  eg16 = jnp.exp(g).astype(jnp.bfloat16)
  denom = jnp.dot(eg16, ones_ref[...], preferred_element_type=jnp.float32)
  o = jnp.dot(eg16[:, :D] * xj16, wgx_ref[...],
              preferred_element_type=jnp.float32)
  o += jnp.dot(eg16[:, D:] * rij16, wgr_ref[...],
               preferred_element_type=jnp.float32)
  msg_ref[...] = o * (1.0 / denom)


def _tc_edge(xj, pf, w65, wpd, bp2, wax, war, ba2, wgx, wgr, ones):
  es = xj.shape[0]
  grid = (es // B_EDGE,)
  full = lambda shape: pl.BlockSpec(shape, lambda i: (0, 0))
  return pl.pallas_call(
      _edge_body,
      grid=grid,
      in_specs=[
          pl.BlockSpec((B_EDGE, D), lambda i: (i, 0)),
          pl.BlockSpec((8, B_EDGE), lambda i: (0, i)),
          full((7, RD + 1)),
          full((1, RD)),
          full((1, RD)),
          full((D, FD)),
          full((RD, FD)),
          full((1, FD)),
          full((D, OD)),
          full((RD, OD)),
          full((FD, 1)),
      ],
      out_specs=pl.BlockSpec((B_EDGE, OD), lambda i: (i, 0)),
      out_shape=jax.ShapeDtypeStruct((es, OD), jnp.float32),
  )(xj, pf, w65, wpd, bp2, wax, war, ba2, wgx, wgr, ones)


B_OUT = 2000


def _out_body(*refs):
  ps = refs[:-2]
  bg_ref = refs[-2]
  out_ref = refs[-1]
  acc = ps[0][...]
  for r in ps[1:]:
    acc += r[...]
  out_ref[...] = jnp.maximum(acc + bg_ref[...], 0.0)


def _tc_out(partials, bg2):
  grid = (N // B_OUT,)
  return pl.pallas_call(
      _out_body,
      grid=grid,
      in_specs=[pl.BlockSpec((B_OUT, OD), lambda i: (i, 0))
                for _ in partials] + [pl.BlockSpec((1, OD), lambda i: (0, 0))],
      out_specs=pl.BlockSpec((B_OUT, OD), lambda i: (i, 0)),
      out_shape=jax.ShapeDtypeStruct((N, OD), jnp.float32),
  )(*partials, bg2)


def kernel(x, pos, edge_index, Wp, bp, Wa, ba, Wg, bg):
  src = edge_index[0]
  dst = edge_index[1]
  pos4 = jnp.pad(pos, ((0, 0), (0, 1)))
  ei_pad = jnp.pad(edge_index.reshape(2, NCH, KC),
                   ((0, 0), (0, NCMAX * NW - NCH), (0, 0)))
  ei_w = ei_pad.reshape(2, NCMAX, NW, KC).transpose(0, 2, 1, 3)
  srcw = ei_w[0]
  dstw = ei_w[1]

  w6 = jnp.concatenate([Wp[0:3] + Wp[6:9], Wp[3:6] - Wp[6:9]], axis=0)
  d2col = jnp.concatenate([jnp.zeros((6, 1), jnp.float32),
                           jnp.ones((1, 1), jnp.float32)], axis=0)
  w65 = jnp.concatenate([jnp.pad(w6, ((0, 1), (0, 0))), d2col],
                        axis=1)
  wpd = Wp[9:10]
  bf = jnp.bfloat16
  pos4f = pos4.reshape(-1)
  x16 = x.astype(bf)

  q0 = jnp.zeros((N, OD), jnp.float32)
  q1 = q0
  for s in range(NSLICE):
    xj, pf = _sc_gather(x, pos4f, srcw, dstw, s)
    msg = _tc_edge(xj, pf, w65, wpd, bp.reshape(1, RD),
                   Wa[:D].astype(bf), Wa[D:].astype(bf), ba.reshape(1, FD),
                   Wg[:D].astype(bf), Wg[D:].astype(bf),
                   jnp.ones((FD, 1), bf))
    q0, q1 = _sc_scatter(msg, dstw, q0, q1, s)

  return _tc_out([q0, q1], bg.reshape(1, OD))
